# baseline (speedup 1.0000x reference)
import functools

import jax
import jax.numpy as jnp
from jax import lax
from jax.experimental import pallas as pl
from jax.experimental.pallas import tpu as pltpu
from jax.experimental.pallas import tpu_sc as plsc

F = 26
OUT = 16
B = 16384
NW = 32
BPT = B // NW
C = 128
NCHUNK = BPT // C
CF = C * F
HI = F - 16

N = 1000012
NMAIN = 1000008
RK = 256


def _build_ids(x, off2):
    mesh = plsc.VectorSubcoreMesh(core_axis_name="c", subcore_axis_name="s")

    @functools.partial(
        pl.kernel,
        out_type=jax.ShapeDtypeStruct((B * F,), jnp.int32),
        mesh=mesh,
        compiler_params=pltpu.CompilerParams(use_tc_tiling_on_sc=True),
        scratch_types=[
            pltpu.VMEM((2, OUT), jnp.int32),
            pltpu.VMEM((BPT, F), jnp.int32),
            pltpu.VMEM((BPT * F,), jnp.int32),
        ],
    )
    def ka(x_hbm, off2_hbm, ids_hbm, off2_v, xv, obuf):
        cid = lax.axis_index("c")
        sid = lax.axis_index("s")
        wid = sid * 2 + cid
        base = wid * BPT
        pltpu.sync_copy(off2_hbm, off2_v)
        pltpu.sync_copy(x_hbm.at[pl.ds(base, BPT), :], xv)
        off_lo = off2_v[0, :]
        off_hi = off2_v[1, :]

        def row_body(j, carry):
            obuf[pl.ds(j * F, 16)] = xv[j, pl.ds(0, 16)] + off_lo
            obuf[pl.ds(j * F + HI, 16)] = xv[j, pl.ds(HI, 16)] + off_hi
            return carry

        lax.fori_loop(0, BPT, row_body, 0)
        pltpu.sync_copy(obuf, ids_hbm.at[pl.ds(base * F, BPT * F)])

    return ka(x, off2)


def _depad_weight(weight, tail):
    mesh = plsc.VectorSubcoreMesh(core_axis_name="c", subcore_axis_name="s")

    @functools.partial(
        pl.kernel,
        out_type=jax.ShapeDtypeStruct((N * OUT,), jnp.float32),
        mesh=mesh,
        compiler_params=pltpu.CompilerParams(use_tc_tiling_on_sc=True),
        scratch_types=[
            pltpu.VMEM((2, RK, OUT), jnp.float32),
            pltpu.VMEM((2, RK * OUT), jnp.float32),
            pltpu.VMEM((8, OUT), jnp.float32),
            pltpu.VMEM((8 * OUT,), jnp.float32),
        ],
    )
    def kw(w_hbm, tail_hbm, out_hbm, vbuf, obuf, tv, tb):
        cid = lax.axis_index("c")
        sid = lax.axis_index("s")
        wid = sid * 2 + cid

        def do_chunk(r0, pb):
            pltpu.sync_copy(w_hbm.at[pl.ds(r0, RK), :], vbuf.at[pb])

            def row_body(j, carry):
                obuf[pb, pl.ds(j * OUT, OUT)] = vbuf[pb, j, :]
                return carry

            lax.fori_loop(0, RK, row_body, 0)
            pltpu.sync_copy(obuf.at[pb], out_hbm.at[pl.ds(r0 * OUT, RK * OUT)])

        nchunk = NMAIN // RK
        rem = NMAIN - nchunk * RK

        def loop_body(i, carry):
            c = i * NW + wid
            @pl.when(c < nchunk)
            def _():
                do_chunk(c * RK, 0)
            return carry

        lax.fori_loop(0, (nchunk + NW - 1) // NW, loop_body, 0)

        @pl.when(wid == 0)
        def _():
            pltpu.sync_copy(w_hbm.at[pl.ds(nchunk * RK, rem), :],
                            vbuf.at[1, pl.ds(0, rem), :])

            def row_body(j, carry):
                obuf[1, pl.ds(j * OUT, OUT)] = vbuf[1, j, :]
                return carry

            lax.fori_loop(0, rem, row_body, 0)
            pltpu.sync_copy(obuf.at[1, pl.ds(0, rem * OUT)],
                            out_hbm.at[pl.ds(nchunk * RK * OUT, rem * OUT)])

        @pl.when(wid == 1)
        def _():
            pltpu.sync_copy(tail_hbm.at[pl.ds(4, 4), :], tv.at[pl.ds(0, 4), :])

            def row_body(j, carry):
                tb[pl.ds(j * OUT, OUT)] = tv[j, :]
                return carry

            lax.fori_loop(0, N - NMAIN, row_body, 0)
            tn = (N - NMAIN) * OUT
            pltpu.sync_copy(tb.at[pl.ds(0, tn)],
                            out_hbm.at[pl.ds(NMAIN * OUT, tn)])

    return kw(weight, tail)


def _gather_sum(ids, wflat, bias):
    mesh = plsc.VectorSubcoreMesh(core_axis_name="c", subcore_axis_name="s")

    @functools.partial(
        pl.kernel,
        out_type=jax.ShapeDtypeStruct((B, OUT), jnp.float32),
        mesh=mesh,
        compiler_params=pltpu.CompilerParams(use_tc_tiling_on_sc=False),
        scratch_types=[
            pltpu.VMEM((OUT,), jnp.float32),
            pltpu.VMEM((2, CF), jnp.int32),
            pltpu.VMEM((2, CF, OUT), jnp.float32),
            pltpu.VMEM((C, OUT), jnp.float32),
            pltpu.SemaphoreType.DMA,
            pltpu.SemaphoreType.DMA,
        ],
    )
    def kb(ids_hbm, w_hbm, bias_hbm, out_hbm,
           bias_v, idx_v, gbuf, outb, sem0, sem1):
        cid = lax.axis_index("c")
        sid = lax.axis_index("s")
        wid = sid * 2 + cid
        tbase = wid * BPT
        sems = (sem0, sem1)

        pltpu.sync_copy(bias_hbm, bias_v)
        bias_vec = bias_v[:]

        def stage_in(ci, pb):
            base = tbase + ci * C
            pltpu.sync_copy(ids_hbm.at[pl.ds(base * F, CF)], idx_v.at[pb])
            return [
                pltpu.async_copy(w_hbm.at[idx_v.at[pb, pl.ds(g * C, C)]],
                                 gbuf.at[pb, pl.ds(g * C, C), :], sems[pb])
                for g in range(F)
            ]

        def stage_out(ci, pb, descs):
            for dsc in descs:
                dsc.wait()

            def row_body(j, carry):
                rbase = j * F
                acc = bias_vec
                for f in range(F):
                    acc = acc + gbuf[pb, rbase + f, :]
                outb[j, :] = acc
                return carry

            lax.fori_loop(0, C, row_body, 0)
            base = tbase + ci * C
            pltpu.sync_copy(outb, out_hbm.at[pl.ds(base, C), :])

        descs = stage_in(0, 0)
        for ci in range(NCHUNK):
            nxt = None
            if ci + 1 < NCHUNK:
                nxt = stage_in(ci + 1, (ci + 1) % 2)
            stage_out(ci, ci % 2, descs)
            descs = nxt

    return kb(ids, wflat, bias)


def kernel(x, weight, bias, offset):
    offi = offset.astype(jnp.int32)
    off2 = jnp.stack([offi[0:16], offi[HI:F]])
    ids = _build_ids(x, off2)
    tail = lax.slice(weight, (N - 8, 0), (N, OUT))
    wflat = _depad_weight(weight, tail)
    return _gather_sum(ids, wflat.reshape(N, OUT), bias.astype(jnp.float32))


# speedup vs baseline: 1.2723x; 1.2723x over previous
import functools

import jax
import jax.numpy as jnp
from jax import lax
from jax.experimental import pallas as pl
from jax.experimental.pallas import tpu as pltpu
from jax.experimental.pallas import tpu_sc as plsc

F = 26
OUT = 16
B = 16384
NW = 32
BPT = B // NW
C = 128
NCHUNK = BPT // C
CF = C * F
HI = F - 16

N = 1000012
RPT = 31248
RK = 336
KCH = RPT // RK
NMAIN = NW * RPT
LEFT = 1000008 - NMAIN


def _prep(x, off2, weight, tail):
    mesh = plsc.VectorSubcoreMesh(core_axis_name="c", subcore_axis_name="s")

    @functools.partial(
        pl.kernel,
        out_type=(
            jax.ShapeDtypeStruct((B * F,), jnp.int32),
            jax.ShapeDtypeStruct((N * OUT,), jnp.float32),
        ),
        mesh=mesh,
        compiler_params=pltpu.CompilerParams(use_tc_tiling_on_sc=True),
        scratch_types=[
            pltpu.VMEM((2, OUT), jnp.int32),
            pltpu.VMEM((C, F), jnp.int32),
            pltpu.VMEM((CF,), jnp.int32),
            pltpu.VMEM((RK, OUT), jnp.float32),
            pltpu.VMEM((RK, OUT), jnp.float32),
            pltpu.VMEM((RK * OUT,), jnp.float32),
            pltpu.VMEM((RK * OUT,), jnp.float32),
            pltpu.SemaphoreType.DMA,
            pltpu.SemaphoreType.DMA,
            pltpu.SemaphoreType.DMA,
            pltpu.SemaphoreType.DMA,
        ],
    )
    def kp(x_hbm, off2_hbm, w_hbm, tail_hbm, ids_hbm, wf_hbm,
           off2_v, xv, iv, vb0, vb1, ob0, ob1, si0, si1, so0, so1):
        cid = lax.axis_index("c")
        sid = lax.axis_index("s")
        wid = sid * 2 + cid

        pltpu.sync_copy(off2_hbm, off2_v)
        off_lo = off2_v[0, :]
        off_hi = off2_v[1, :]
        for ci in range(NCHUNK):
            base = wid * BPT + ci * C
            pltpu.sync_copy(x_hbm.at[pl.ds(base, C), :], xv)

            def row_body(j, carry):
                iv[pl.ds(j * F, 16)] = xv[j, pl.ds(0, 16)] + off_lo
                iv[pl.ds(j * F + HI, 16)] = xv[j, pl.ds(HI, 16)] + off_hi
                return carry

            lax.fori_loop(0, C, row_body, 0)
            pltpu.sync_copy(iv, ids_hbm.at[pl.ds(base * F, CF)])

        r00 = wid * RPT
        vbs = (vb0, vb1)
        obs = (ob0, ob1)
        sis = (si0, si1)
        sos = (so0, so1)

        def fire_in(c, pb):
            pltpu.async_copy(w_hbm.at[pl.ds(r00 + c * RK, RK), :],
                             vbs[pb], sis[pb])

        def wait_in(pb):
            pltpu.make_async_copy(w_hbm.at[pl.ds(r00, RK), :],
                                  vbs[pb], sis[pb]).wait()

        def fire_out(c, pb):
            pltpu.async_copy(obs[pb],
                             wf_hbm.at[pl.ds((r00 + c * RK) * OUT, RK * OUT)],
                             sos[pb])

        def wait_out(pb):
            pltpu.make_async_copy(obs[pb],
                                  wf_hbm.at[pl.ds(r00 * OUT, RK * OUT)],
                                  sos[pb]).wait()

        def compact(pb):
            def cbody(jj, carry):
                for r in range(8):
                    j = jj * 8 + r
                    obs[pb][pl.ds(j * OUT, OUT)] = vbs[pb][j, :]
                return carry

            lax.fori_loop(0, RK // 8, cbody, 0)

        fire_in(0, 0)
        fire_in(1, 1)

        def pipe_body(i, carry):
            c0 = i * 2
            wait_in(0)

            @pl.when(i > 0)
            def _():
                wait_out(0)

            compact(0)
            fire_in(c0 + 2, 0)
            fire_out(c0, 0)
            wait_in(1)

            @pl.when(i > 0)
            def _():
                wait_out(1)

            compact(1)

            @pl.when(c0 + 3 < KCH)
            def _():
                fire_in(c0 + 3, 1)

            fire_out(c0 + 1, 1)
            return carry

        lax.fori_loop(0, (KCH - 1) // 2, pipe_body, 0)
        wait_in(0)
        wait_out(0)
        compact(0)
        fire_out(KCH - 1, 0)
        wait_out(1)
        wait_out(0)

        @pl.when(wid == 0)
        def _():
            pltpu.sync_copy(w_hbm.at[pl.ds(NMAIN, LEFT), :],
                            vb0.at[pl.ds(0, LEFT), :])

            def rbody(j, carry):
                ob0[pl.ds(j * OUT, OUT)] = vb0[j, :]
                return carry

            lax.fori_loop(0, LEFT, rbody, 0)
            pltpu.sync_copy(ob0.at[pl.ds(0, LEFT * OUT)],
                            wf_hbm.at[pl.ds(NMAIN * OUT, LEFT * OUT)])

        @pl.when(wid == 1)
        def _():
            pltpu.sync_copy(tail_hbm.at[pl.ds(4, 4), :], vb1.at[pl.ds(0, 4), :])

            def tbody(j, carry):
                ob1[pl.ds(j * OUT, OUT)] = vb1[j, :]
                return carry

            lax.fori_loop(0, 4, tbody, 0)
            pltpu.sync_copy(ob1.at[pl.ds(0, 64)],
                            wf_hbm.at[pl.ds(1000008 * OUT, 64)])

    return kp(x, off2, weight, tail)


def _gather_sum(ids, wflat, bias):
    mesh = plsc.VectorSubcoreMesh(core_axis_name="c", subcore_axis_name="s")

    @functools.partial(
        pl.kernel,
        out_type=jax.ShapeDtypeStruct((B, OUT), jnp.float32),
        mesh=mesh,
        compiler_params=pltpu.CompilerParams(use_tc_tiling_on_sc=False),
        scratch_types=[
            pltpu.VMEM((OUT,), jnp.float32),
            pltpu.VMEM((2, CF), jnp.int32),
            pltpu.VMEM((2, CF, OUT), jnp.float32),
            pltpu.VMEM((C, OUT), jnp.float32),
            pltpu.SemaphoreType.DMA,
            pltpu.SemaphoreType.DMA,
        ],
    )
    def kb(ids_hbm, w_hbm, bias_hbm, out_hbm,
           bias_v, idx_v, gbuf, outb, sem0, sem1):
        cid = lax.axis_index("c")
        sid = lax.axis_index("s")
        wid = sid * 2 + cid
        tbase = wid * BPT
        sems = (sem0, sem1)

        pltpu.sync_copy(bias_hbm, bias_v)
        bias_vec = bias_v[:]

        def stage_in(ci, pb):
            base = tbase + ci * C
            pltpu.sync_copy(ids_hbm.at[pl.ds(base * F, CF)], idx_v.at[pb])
            return [
                pltpu.async_copy(w_hbm.at[idx_v.at[pb, pl.ds(g * C, C)]],
                                 gbuf.at[pb, pl.ds(g * C, C), :], sems[pb])
                for g in range(F)
            ]

        def stage_out(ci, pb, descs):
            for dsc in descs:
                dsc.wait()

            def row_body(j, carry):
                rbase = j * F
                acc = bias_vec
                for f in range(F):
                    acc = acc + gbuf[pb, rbase + f, :]
                outb[j, :] = acc
                return carry

            lax.fori_loop(0, C, row_body, 0)
            base = tbase + ci * C
            pltpu.sync_copy(outb, out_hbm.at[pl.ds(base, C), :])

        descs = stage_in(0, 0)
        for ci in range(NCHUNK):
            nxt = None
            if ci + 1 < NCHUNK:
                nxt = stage_in(ci + 1, (ci + 1) % 2)
            stage_out(ci, ci % 2, descs)
            descs = nxt

    return kb(ids, wflat, bias)


def kernel(x, weight, bias, offset):
    offi = offset.astype(jnp.int32)
    off2 = jnp.stack([offi[0:16], offi[HI:F]])
    tail = lax.slice(weight, (N - 8, 0), (N, OUT))
    ids, wflat = _prep(x, off2, weight, tail)
    return _gather_sum(ids, wflat.reshape(N, OUT), bias.astype(jnp.float32))


# speedup vs baseline: 1.4476x; 1.1378x over previous
import functools

import jax
import jax.numpy as jnp
from jax import lax
from jax.experimental import pallas as pl
from jax.experimental.pallas import tpu as pltpu
from jax.experimental.pallas import tpu_sc as plsc

F = 26
OUT = 16
B = 16384
NW = 32
BPT = B // NW
C = 128
NCHUNK = BPT // C
NV = C // 16


def _field_linear_sc(xt, weight, off2, bias):
    mesh = plsc.VectorSubcoreMesh(core_axis_name="c", subcore_axis_name="s")

    @functools.partial(
        pl.kernel,
        out_type=jax.ShapeDtypeStruct((B, OUT), jnp.float32),
        mesh=mesh,
        compiler_params=pltpu.CompilerParams(use_tc_tiling_on_sc=False),
        scratch_types=[
            pltpu.VMEM((F, OUT), jnp.int32),
            pltpu.VMEM((OUT,), jnp.float32),
            pltpu.VMEM((2, F, C), jnp.int32),
            pltpu.VMEM((2, F, C), jnp.int32),
            pltpu.VMEM((2, F, C, OUT), jnp.float32),
            pltpu.VMEM((C, OUT), jnp.float32),
            pltpu.SemaphoreType.DMA,
            pltpu.SemaphoreType.DMA,
        ],
    )
    def k(xt_hbm, w_hbm, off2_hbm, bias_hbm, out_hbm,
          off2_v, bias_v, xv, idx_v, gbuf, outb, sem0, sem1):
        cid = lax.axis_index("c")
        sid = lax.axis_index("s")
        wid = sid * 2 + cid
        tbase = wid * BPT
        sems = (sem0, sem1)

        pltpu.sync_copy(off2_hbm, off2_v)
        pltpu.sync_copy(bias_hbm, bias_v)
        bias_vec = bias_v[:]

        def stage_in(ci, pb):
            base = tbase + ci * C
            pltpu.sync_copy(xt_hbm.at[:, pl.ds(base, C)], xv.at[pb])

            def vbody(j, carry):
                s = pl.ds(j * 16, 16)
                for f in range(F):
                    idx_v[pb, f, s] = xv[pb, f, s] + off2_v[f, :]
                return carry

            lax.fori_loop(0, NV, vbody, 0)
            return [
                pltpu.async_copy(w_hbm.at[idx_v.at[pb, f]], gbuf.at[pb, f],
                                 sems[pb])
                for f in range(F)
            ]

        def stage_out(ci, pb, descs):
            for dsc in descs:
                dsc.wait()

            def row_body(j, carry):
                acc = bias_vec
                for f in range(F):
                    acc = acc + gbuf[pb, f, j, :]
                outb[j, :] = acc
                return carry

            lax.fori_loop(0, C, row_body, 0)
            base = tbase + ci * C
            pltpu.sync_copy(outb, out_hbm.at[pl.ds(base, C), :])

        descs = stage_in(0, 0)
        for ci in range(NCHUNK):
            nxt = None
            if ci + 1 < NCHUNK:
                nxt = stage_in(ci + 1, (ci + 1) % 2)
            stage_out(ci, ci % 2, descs)
            descs = nxt

    return k(xt, weight, off2, bias)


def kernel(x, weight, bias, offset):
    off2 = jnp.broadcast_to(offset.astype(jnp.int32)[:, None], (F, OUT))
    return _field_linear_sc(x.T, weight, off2, bias.astype(jnp.float32))
